# Initial kernel scaffold; baseline (speedup 1.0000x reference)
import jax
import jax.numpy as jnp
from jax.experimental import pallas as pl


def kernel(x, edge_index, edge_type, R):
    raise NotImplementedError("write your pallas kernel here")


# speedup vs baseline: 2.8056x; 2.8056x over previous
import functools

import jax
import jax.numpy as jnp
from jax import lax
from jax.experimental import pallas as pl
from jax.experimental.pallas import tpu as pltpu
from jax.experimental.pallas import tpu_sc as plsc

N_NODES = 10000
DIM = 128
HD = DIM // 2
N_EDGES = 320000
N_REL = 964

NC = 2
NS = 16
NW = NC * NS
L = 16

CH = N_EDGES // NW
TPAD = 976
NV = TPAD // L
B = 80
NB = CH // B

_mesh = plsc.VectorSubcoreMesh(core_axis_name="c", subcore_axis_name="s")


def _wid():
    return lax.axis_index("c") * NS + lax.axis_index("s")


@functools.partial(
    pl.kernel,
    out_type=jax.ShapeDtypeStruct((NW, TPAD), jnp.int32),
    mesh=_mesh,
    compiler_params=pltpu.CompilerParams(use_tc_tiling_on_sc=False,
                                         needs_layout_passes=False),
    scratch_types=[
        pltpu.VMEM((CH,), jnp.int32),
        pltpu.VMEM((TPAD,), jnp.int32),
    ],
)
def _hist_kernel(et_hbm, hist_hbm, et_v, h1d):
    wid = _wid()
    pltpu.sync_copy(et_hbm.at[pl.ds(wid * CH, CH)], et_v)

    zero16 = jnp.zeros((L,), jnp.int32)

    def zero_body(j, _):
        h1d[pl.ds(j * L, L)] = zero16
        return 0

    lax.fori_loop(0, NV, zero_body, 0)

    ones = jnp.ones((L,), jnp.int32)

    def hist_body(g, _):
        tv = et_v[pl.ds(g * L, L)]
        plsc.addupdate_scatter(h1d, [tv], ones)
        return 0

    lax.fori_loop(0, CH // L, hist_body, 0)
    pltpu.sync_copy(h1d, hist_hbm.at[wid])


@functools.partial(
    pl.kernel,
    out_type=jax.ShapeDtypeStruct((N_EDGES,), jnp.float32),
    mesh=_mesh,
    compiler_params=pltpu.CompilerParams(use_tc_tiling_on_sc=False,
                                         needs_layout_passes=False),
    scratch_types=[
        pltpu.VMEM((N_REL, HD), jnp.int32),
        pltpu.VMEM((8, TPAD), jnp.int32),
        pltpu.VMEM((TPAD,), jnp.int32),
        pltpu.VMEM((TPAD,), jnp.int32),
        pltpu.VMEM((CH,), jnp.int32),
        pltpu.VMEM((CH,), jnp.int32),
        pltpu.VMEM((CH,), jnp.int32),
        pltpu.VMEM((2, B, HD), jnp.int32),
        pltpu.VMEM((2, B, HD), jnp.int32),
        pltpu.VMEM((2, B), jnp.int32),
        pltpu.VMEM((2, B), jnp.float32),
        pltpu.VMEM((3 * L,), jnp.int32),
        pltpu.SemaphoreType.DMA,
        pltpu.SemaphoreType.DMA,
        pltpu.SemaphoreType.DMA,
        pltpu.SemaphoreType.DMA,
        pltpu.SemaphoreType.DMA,
        pltpu.SemaphoreType.DMA,
    ],
)
def _main_kernel(x_hbm, left_hbm, right_hbm, et_hbm, r_hbm, hist_hbm, out_hbm,
                 rels_v, row8_v, tot_v, base_v, lid_all, rid_all, tid_all,
                 xl2, xr2, pos2, sc2, tbuf,
                 gsem0, gsem1, gsem2, gsem3, ssem0, ssem1):
    wid = _wid()
    pltpu.sync_copy(r_hbm, rels_v)
    chunk0 = wid * CH
    pltpu.sync_copy(left_hbm.at[pl.ds(chunk0, CH)], lid_all)
    pltpu.sync_copy(right_hbm.at[pl.ds(chunk0, CH)], rid_all)
    pltpu.sync_copy(et_hbm.at[pl.ds(chunk0, CH)], tid_all)
    wid_v = jnp.zeros((L,), jnp.int32) + wid
    zero16 = jnp.zeros((L,), jnp.int32)

    def zero_body(j, _):
        tot_v[pl.ds(j * L, L)] = zero16
        base_v[pl.ds(j * L, L)] = zero16
        return 0

    lax.fori_loop(0, NV, zero_body, 0)

    for piece in range(NW // 8):
        pltpu.sync_copy(hist_hbm.at[pl.ds(piece * 8, 8)], row8_v)

        def pc_body(j, _):
            js = pl.ds(j * L, L)
            t = tot_v[js]
            p = base_v[js]
            for c8 in range(8):
                c = piece * 8 + c8
                v = row8_v[c8, js]
                t = t + v
                p = jnp.where(jnp.full((L,), c, jnp.int32) < wid_v, p + v, p)
            tot_v[js] = t
            base_v[js] = p
            return 0

        lax.fori_loop(0, NV, pc_body, 0)

    def scan_body(j, carry):
        js = pl.ds(j * L, L)
        tot = tot_v[js]
        inc = plsc.cumsum(tot)
        base_v[js] = base_v[js] + (inc - tot) + carry
        return carry + jnp.sum(tot)

    lax.fori_loop(0, NV, scan_body, jnp.int32(0))

    tbuf[pl.ds(0, L)] = jnp.full((L,), -1, jnp.int32)
    tbuf[pl.ds(2 * L, L)] = jnp.full((L,), -2, jnp.int32)
    lanes = lax.iota(jnp.int32, L)
    ones = jnp.ones((L,), jnp.int32)

    gsems = ((gsem0, gsem1), (gsem2, gsem3))
    ssems = (ssem0, ssem1)

    def g_start(b, s):
        i0 = pl.ds(b * B, B)
        pltpu.async_copy(x_hbm.at[lid_all.at[i0]], xl2.at[s], gsems[s][0])
        pltpu.async_copy(x_hbm.at[rid_all.at[i0]], xr2.at[s], gsems[s][1])

    def g_wait(s):
        i0 = pl.ds(0, B)
        pltpu.make_async_copy(x_hbm.at[lid_all.at[i0]], xl2.at[s],
                              gsems[s][0]).wait()
        pltpu.make_async_copy(x_hbm.at[rid_all.at[i0]], xr2.at[s],
                              gsems[s][1]).wait()

    def s_drain(s):
        pltpu.make_async_copy(sc2.at[s], out_hbm.at[pos2.at[s]],
                              ssems[s]).wait()

    def compute(b, s):
        for g in range(B // L):
            tv = tid_all[pl.ds(b * B + g * L, L)]
            tbuf[pl.ds(L, L)] = tv
            rank = zero16
            for k in range(1, L):
                shm = tbuf[pl.ds(L - k, L)]
                rank = rank + jnp.where(shm == tv, ones, zero16)
            gb = plsc.load_gather(base_v, [tv])
            pos2[s, pl.ds(g * L, L)] = gb + rank
            plsc.addupdate_scatter(base_v, [tv], ones)

        for g in range(B // L):
            rows = lanes + (g * L)
            tv = tid_all[pl.ds(b * B + g * L, L)]

            @plsc.parallel_loop(0, HD, unroll=2,
                                carry=(jnp.zeros((L,), jnp.float32),
                                       jnp.zeros((L,), jnp.float32)))
            def acc_fn(c, acc):
                acc_a, acc_b = acc
                cols = zero16 + c
                lw = plsc.load_gather(xl2.at[s], [rows, cols])
                xw = plsc.load_gather(xr2.at[s], [rows, cols])
                rw = plsc.load_gather(rels_v, [tv, cols])
                lb = plsc.bitcast(lw, jnp.bfloat16)
                xb = plsc.bitcast(xw, jnp.bfloat16)
                rb = plsc.bitcast(rw, jnp.bfloat16)
                prod = lb * xb
                pa, pb = plsc.unpack(prod, format=plsc.PackFormat.INTERLEAVED)
                ra, rb2 = plsc.unpack(rb, format=plsc.PackFormat.INTERLEAVED)
                return acc_a + pa * ra, acc_b + pb * rb2

            acc_a, acc_b = acc_fn
            tot = acc_a + acc_b
            sc2[s, pl.ds(g * L, L)] = 1.0 / (1.0 + jnp.exp(-tot))

        pltpu.async_copy(sc2.at[s], out_hbm.at[pos2.at[s]], ssems[s])

    g_start(0, 0)

    def body(h, _):
        b0 = 2 * h

        @pl.when(h > 0)
        def _():
            s_drain(0)
            s_drain(1)

        g_start(b0 + 1, 1)
        g_wait(0)
        compute(b0, 0)
        g_start(b0 + 2, 0)
        g_wait(1)
        compute(b0 + 1, 1)
        return 0

    lax.fori_loop(0, (NB - 1) // 2, body, 0)
    s_drain(0)
    s_drain(1)
    g_wait(0)
    compute(NB - 1, 0)
    s_drain(0)


def _pack_rows(a):
    b = a.astype(jnp.bfloat16)
    return jax.lax.bitcast_convert_type(
        b.reshape(a.shape[0], a.shape[1] // 2, 2), jnp.int32)


def kernel(x, edge_index, edge_type, R):
    left = edge_index[0]
    right = edge_index[1]
    hist = _hist_kernel(edge_type)
    return _main_kernel(_pack_rows(x), left, right, edge_type,
                        _pack_rows(R), hist)


# speedup vs baseline: 6.2280x; 2.2199x over previous
import functools

import jax
import jax.numpy as jnp
from jax import lax
from jax.experimental import pallas as pl
from jax.experimental.pallas import tpu as pltpu
from jax.experimental.pallas import tpu_sc as plsc

N_NODES = 10000
DIM = 128
HD = DIM // 2
N_EDGES = 320000
N_REL = 964

NC = 2
NS = 16
NW = NC * NS
L = 16

CH = N_EDGES // NW
TPAD = 976
NV = TPAD // L
B = 80
NB = CH // B

_mesh = plsc.VectorSubcoreMesh(core_axis_name="c", subcore_axis_name="s")


def _wid():
    return lax.axis_index("c") * NS + lax.axis_index("s")


@functools.partial(
    pl.kernel,
    out_type=jax.ShapeDtypeStruct((NW, TPAD), jnp.int32),
    mesh=_mesh,
    compiler_params=pltpu.CompilerParams(use_tc_tiling_on_sc=False,
                                         needs_layout_passes=False),
    scratch_types=[
        pltpu.VMEM((CH,), jnp.int32),
        pltpu.VMEM((TPAD,), jnp.int32),
    ],
)
def _hist_kernel(et_hbm, hist_hbm, et_v, h1d):
    wid = _wid()
    pltpu.sync_copy(et_hbm.at[pl.ds(wid * CH, CH)], et_v)

    zero16 = jnp.zeros((L,), jnp.int32)

    def zero_body(j, _):
        h1d[pl.ds(j * L, L)] = zero16
        return 0

    lax.fori_loop(0, NV, zero_body, 0)

    ones = jnp.ones((L,), jnp.int32)

    def hist_body(g, _):
        tv = et_v[pl.ds(g * L, L)]
        plsc.addupdate_scatter(h1d, [tv], ones)
        return 0

    lax.fori_loop(0, CH // L, hist_body, 0)
    pltpu.sync_copy(h1d, hist_hbm.at[wid])


@functools.partial(
    pl.kernel,
    out_type=jax.ShapeDtypeStruct((N_EDGES,), jnp.float32),
    mesh=_mesh,
    compiler_params=pltpu.CompilerParams(use_tc_tiling_on_sc=False,
                                         needs_layout_passes=False),
    scratch_types=[
        pltpu.VMEM((N_REL, HD + 1), jnp.int32),
        pltpu.VMEM((B, HD + 1), jnp.int32),
        pltpu.VMEM((B, L), jnp.float32),
        pltpu.VMEM((4, TPAD), jnp.int32),
        pltpu.VMEM((TPAD,), jnp.int32),
        pltpu.VMEM((TPAD,), jnp.int32),
        pltpu.VMEM((2, B), jnp.int32),
        pltpu.VMEM((2, B), jnp.int32),
        pltpu.VMEM((2, B), jnp.int32),
        pltpu.VMEM((2, B, HD), jnp.int32),
        pltpu.VMEM((2, B, HD), jnp.int32),
        pltpu.VMEM((2, B), jnp.int32),
        pltpu.VMEM((2, B), jnp.float32),
        pltpu.VMEM((3 * L,), jnp.int32),
        pltpu.SemaphoreType.DMA,
        pltpu.SemaphoreType.DMA,
        pltpu.SemaphoreType.DMA,
        pltpu.SemaphoreType.DMA,
        pltpu.SemaphoreType.DMA,
        pltpu.SemaphoreType.DMA,
        pltpu.SemaphoreType.DMA,
        pltpu.SemaphoreType.DMA,
    ],
)
def _main_kernel(x_hbm, left_hbm, right_hbm, et_hbm, r_hbm, hist_hbm, out_hbm,
                 rels_v, relx, part_v, row4_v, tot_v, base_v,
                 lidb, ridb, tidb,
                 xl2, xr2, pos2, sc2, tbuf,
                 gsem0, gsem1, gsem2, gsem3, ssem0, ssem1, isem0, isem1):
    wid = _wid()
    pltpu.sync_copy(r_hbm, rels_v)
    chunk0 = wid * CH
    wid_v = jnp.zeros((L,), jnp.int32) + wid
    zero16 = jnp.zeros((L,), jnp.int32)

    def zero_body(j, _):
        tot_v[pl.ds(j * L, L)] = zero16
        base_v[pl.ds(j * L, L)] = zero16
        return 0

    lax.fori_loop(0, NV, zero_body, 0)

    for piece in range(NW // 4):
        pltpu.sync_copy(hist_hbm.at[pl.ds(piece * 4, 4)], row4_v)

        def pc_body(j, _):
            js = pl.ds(j * L, L)
            t = tot_v[js]
            p = base_v[js]
            for c4 in range(4):
                c = piece * 4 + c4
                v = row4_v[c4, js]
                t = t + v
                p = jnp.where(jnp.full((L,), c, jnp.int32) < wid_v, p + v, p)
            tot_v[js] = t
            base_v[js] = p
            return 0

        lax.fori_loop(0, NV, pc_body, 0)

    def scan_body(j, carry):
        js = pl.ds(j * L, L)
        tot = tot_v[js]
        inc = plsc.cumsum(tot)
        base_v[js] = base_v[js] + (inc - tot) + carry
        return carry + jnp.sum(tot)

    lax.fori_loop(0, NV, scan_body, jnp.int32(0))

    tbuf[pl.ds(0, L)] = jnp.full((L,), -1, jnp.int32)
    tbuf[pl.ds(2 * L, L)] = jnp.full((L,), -2, jnp.int32)
    lanes = lax.iota(jnp.int32, L)
    ones = jnp.ones((L,), jnp.int32)

    gsems = ((gsem0, gsem1, isem0), (gsem2, gsem3, isem1))
    ssems = (ssem0, ssem1)

    def id_start(b, s):
        i0 = pl.ds(chunk0 + b * B, B)
        pltpu.async_copy(left_hbm.at[i0], lidb.at[s], gsems[s][2])
        pltpu.async_copy(right_hbm.at[i0], ridb.at[s], gsems[s][2])

    def id_wait(s):
        i0 = pl.ds(0, B)
        pltpu.make_async_copy(left_hbm.at[i0], lidb.at[s], gsems[s][2]).wait()
        pltpu.make_async_copy(right_hbm.at[i0], ridb.at[s], gsems[s][2]).wait()

    def g_start(b, s):
        pltpu.async_copy(x_hbm.at[lidb.at[s]], xl2.at[s], gsems[s][0])
        pltpu.async_copy(x_hbm.at[ridb.at[s]], xr2.at[s], gsems[s][1])
        pltpu.async_copy(et_hbm.at[pl.ds(chunk0 + b * B, B)], tidb.at[s],
                         gsems[s][0])

    def g_wait(s):
        pltpu.make_async_copy(x_hbm.at[lidb.at[s]], xl2.at[s],
                              gsems[s][0]).wait()
        pltpu.make_async_copy(x_hbm.at[ridb.at[s]], xr2.at[s],
                              gsems[s][1]).wait()
        pltpu.make_async_copy(et_hbm.at[pl.ds(0, B)], tidb.at[s],
                              gsems[s][0]).wait()

    def s_drain(s):
        pltpu.make_async_copy(sc2.at[s], out_hbm.at[pos2.at[s]],
                              ssems[s]).wait()

    fifteen = jnp.full((L,), L - 1, jnp.int32)

    def post(b, s):
        for g in range(B // L):
            tv = tidb[s, pl.ds(g * L, L)]
            tbuf[pl.ds(L, L)] = tv
            rank = zero16
            for k in range(1, L):
                shm = tbuf[pl.ds(L - k, L)]
                rank = rank + jnp.where(shm == tv, ones, zero16)
            gb = plsc.load_gather(base_v, [tv])
            pos2[s, pl.ds(g * L, L)] = gb + rank
            plsc.addupdate_scatter(base_v, [tv], ones)

        for g in range(B // L):
            rows = lanes + (g * L)
            tv = tidb[s, pl.ds(g * L, L)]

            @plsc.parallel_loop(0, HD, unroll=2)
            def _(c):
                colc = zero16 + c
                rw = plsc.load_gather(rels_v, [tv, colc])
                plsc.store_scatter(relx, [rows, colc], rw)
        @plsc.parallel_loop(0, B, unroll=2)
        def _(i):
            acc_a = jnp.zeros((L,), jnp.float32)
            acc_b = jnp.zeros((L,), jnp.float32)
            for j in range(DIM // (2 * L)):
                sl = pl.ds(j * L, L)
                lb = plsc.bitcast(xl2[s, i, sl], jnp.bfloat16)
                xb = plsc.bitcast(xr2[s, i, sl], jnp.bfloat16)
                rb = plsc.bitcast(relx[i, sl], jnp.bfloat16)
                prod = lb * xb
                pa, pb = plsc.unpack(prod, format=plsc.PackFormat.INTERLEAVED)
                ra, rb2 = plsc.unpack(rb, format=plsc.PackFormat.INTERLEAVED)
                acc_a = acc_a + pa * ra
                acc_b = acc_b + pb * rb2
            part_v[i] = plsc.cumsum(acc_a + acc_b)

        for g in range(B // L):
            eids = lanes + (g * L)
            tot = plsc.load_gather(part_v, [eids, fifteen])
            sc2[s, pl.ds(g * L, L)] = 1.0 / (1.0 + jnp.exp(-tot))

        pltpu.async_copy(sc2.at[s], out_hbm.at[pos2.at[s]], ssems[s])

    pltpu.sync_copy(left_hbm.at[pl.ds(chunk0, B)], lidb.at[0])
    pltpu.sync_copy(right_hbm.at[pl.ds(chunk0, B)], ridb.at[0])
    g_start(0, 0)
    id_start(1, 1)
    NH = (NB - 1) // 2

    def body(h, _):
        b0 = 2 * h

        @pl.when(h > 0)
        def _():
            s_drain(0)
            s_drain(1)

        id_wait(1)
        g_start(b0 + 1, 1)
        g_wait(0)
        id_start(b0 + 2, 0)
        post(b0, 0)
        id_wait(0)
        g_start(b0 + 2, 0)
        g_wait(1)

        @pl.when(h < NH - 1)
        def _():
            id_start(b0 + 3, 1)

        post(b0 + 1, 1)
        return 0

    lax.fori_loop(0, NH, body, 0)
    s_drain(0)
    s_drain(1)
    g_wait(0)
    post(NB - 1, 0)
    s_drain(0)


def _pack_rows(a):
    b = a.astype(jnp.bfloat16)
    return jax.lax.bitcast_convert_type(
        b.reshape(a.shape[0], a.shape[1] // 2, 2), jnp.int32)


def kernel(x, edge_index, edge_type, R):
    left = edge_index[0]
    right = edge_index[1]
    hist = _hist_kernel(edge_type)
    r_pad = jnp.pad(_pack_rows(R), ((0, 0), (0, 1)))
    return _main_kernel(_pack_rows(x), left, right, edge_type, r_pad, hist)
